# Initial kernel scaffold; baseline (speedup 1.0000x reference)
import jax
import jax.numpy as jnp
from jax.experimental import pallas as pl


def kernel(t, idx, v):
    raise NotImplementedError("write your pallas kernel here")


# speedup vs baseline: 1.3166x; 1.3166x over previous
import jax
import jax.numpy as jnp
from jax import lax
from jax.experimental import pallas as pl
from jax.experimental.pallas import tpu as pltpu
from jax.experimental.pallas import tpu_sc as plsc

M = 1_000_000
D = 64
B = 16384

NC = 2
NS = 16
NW = NC * NS
BPW = B // NW
CH = 64
NCHUNK = BPW // CH

BR = 8000


def _copy_body(t_ref, o_ref):
    o_ref[...] = t_ref[...]


def _sc_scatter_body(idx_hbm, vrows_hbm, out_ref, idx_v, vrows_v, sem):
    wid = lax.axis_index("s") * NC + lax.axis_index("c")
    pltpu.sync_copy(idx_hbm.at[wid], idx_v)
    pltpu.sync_copy(vrows_hbm, vrows_v)
    for j in range(NCHUNK):
        pltpu.async_copy(vrows_v, out_ref.at[idx_v.at[j]], sem).wait()


def kernel(t, idx, v):
    idx3 = idx.astype(jnp.int32).reshape(NW, NCHUNK, CH)
    vrows = jnp.full((CH, D), v, dtype=jnp.float32)

    out = pl.pallas_call(
        _copy_body,
        grid=(M // BR,),
        in_specs=[pl.BlockSpec((BR, D), lambda i: (i, 0))],
        out_specs=pl.BlockSpec((BR, D), lambda i: (i, 0)),
        out_shape=jax.ShapeDtypeStruct((M, D), jnp.float32),
    )(t)

    r = jax.new_ref(out)
    scatter = pl.kernel(
        _sc_scatter_body,
        out_type=(),
        mesh=plsc.VectorSubcoreMesh(core_axis_name="c", subcore_axis_name="s"),
        scratch_types=[
            pltpu.VMEM((NCHUNK, CH), jnp.int32),
            pltpu.VMEM((CH, D), jnp.float32),
            pltpu.SemaphoreType.DMA,
        ],
        compiler_params=pltpu.CompilerParams(use_tc_tiling_on_sc=False),
    )
    scatter(idx3, vrows, r)
    return r[...]
